# baseline (speedup 1.0000x reference)
import jax
import jax.numpy as jnp
from jax.experimental import pallas as pl

CLS_NUM = 1000
D = 512
N_PREFIX = 3
N_SUFFIX = 2
N_CTX = N_PREFIX + N_SUFFIX
N_CLS_TOK = 3
CTX_LEN = 77
PAD_SIZE = CTX_LEN - (N_CTX + N_CLS_TOK + 2)
PAD_LEN = 75

SLAB = 11
NSTEPS = CTX_LEN // SLAB


def _body(ctx_ref, sot_ref, eot_ref, pad_ref, cls_ref, out_ref):
    s = pl.program_id(0)

    for r in range(SLAB):
        t = s * SLAB + r

        def bcast(row, r=r):
            out_ref[r, :, :] = jnp.broadcast_to(row.reshape(1, D),
                                                (CLS_NUM, D))

        @pl.when(t == 0)
        def _(r=r):
            bcast(sot_ref[0, 0:1, :])

        @pl.when(jnp.logical_and(t >= 1, t < 4))
        def _(r=r, t=t):
            bcast(ctx_ref[0, pl.ds(jnp.clip(t - 1, 0, N_CTX - 1), 1), :])

        @pl.when(jnp.logical_and(t >= 4, t < 7))
        def _(r=r, t=t):
            j = jnp.clip(t - 4, 0, N_CLS_TOK - 1)
            out_ref[r, :, :] = cls_ref[:, pl.ds(j, 1), :].reshape(CLS_NUM, D)

        @pl.when(jnp.logical_and(t >= 7, t < 9))
        def _(r=r, t=t):
            bcast(ctx_ref[0, pl.ds(jnp.clip(t - 4, 0, N_CTX - 1), 1), :])

        @pl.when(t == 9)
        def _(r=r):
            bcast(eot_ref[0, 0:1, :])

        @pl.when(t >= 10)
        def _(r=r, t=t):
            bcast(pad_ref[0, pl.ds(jnp.clip(t - 10, 0, PAD_LEN - 1), 1), :])


def kernel(ctx, emb_sot, emb_cls, emb_eot, emb_pad):
    tposed = pl.pallas_call(
        _body,
        grid=(NSTEPS,),
        in_specs=[
            pl.BlockSpec((1, N_CTX, D), lambda t: (0, 0, 0)),
            pl.BlockSpec((1, 1, D), lambda t: (0, 0, 0)),
            pl.BlockSpec((1, 1, D), lambda t: (0, 0, 0)),
            pl.BlockSpec((1, PAD_LEN, D), lambda t: (0, 0, 0)),
            pl.BlockSpec((CLS_NUM, N_CLS_TOK, D), lambda t: (0, 0, 0)),
        ],
        out_specs=pl.BlockSpec((SLAB, CLS_NUM, D), lambda t: (t, 0, 0)),
        out_shape=jax.ShapeDtypeStruct((CTX_LEN, CLS_NUM, D), jnp.float32),
    )(ctx, emb_sot, emb_eot, emb_pad, emb_cls)
    return jnp.transpose(tposed, (1, 0, 2))


# speedup vs baseline: 1.0547x; 1.0547x over previous
import jax
import jax.numpy as jnp
from jax.experimental import pallas as pl
from jax.experimental.pallas import tpu as pltpu

CLS_NUM = 1000
D = 512
N_PREFIX = 3
N_SUFFIX = 2
N_CTX = N_PREFIX + N_SUFFIX
N_CLS_TOK = 3
CTX_LEN = 77
PAD_SIZE = CTX_LEN - (N_CTX + N_CLS_TOK + 2)
PAD_LEN = 75

SLAB = 7
NSTEPS = CTX_LEN // SLAB


def _body(ctx_ref, sot_ref, eot_ref, pad_ref, cls_hbm, out_ref, cls_v, sem):
    s = pl.program_id(0)
    tb = (s + 1) % NSTEPS

    @pl.when(s == 0)
    def _fetch():
        pltpu.make_async_copy(cls_hbm, cls_v, sem).start()

    @pl.when(s == NSTEPS - 1)
    def _land():
        pltpu.make_async_copy(cls_hbm, cls_v, sem).wait()

    for r in range(SLAB):
        t = tb * SLAB + r

        def bcast(row, r=r):
            out_ref[r, :, :] = jnp.broadcast_to(row.reshape(1, D),
                                                (CLS_NUM, D))

        @pl.when(t == 0)
        def _(r=r):
            bcast(sot_ref[0, 0:1, :])

        @pl.when(jnp.logical_and(t >= 1, t < 4))
        def _(r=r, t=t):
            bcast(ctx_ref[0, pl.ds(jnp.clip(t - 1, 0, N_CTX - 1), 1), :])

        @pl.when(jnp.logical_and(t >= 4, t < 7))
        def _(r=r, t=t):
            j = jnp.clip(t - 4, 0, N_CLS_TOK - 1)
            out_ref[r, :, :] = cls_v[:, pl.ds(j, 1), :].reshape(CLS_NUM, D)

        @pl.when(jnp.logical_and(t >= 7, t < 9))
        def _(r=r, t=t):
            bcast(ctx_ref[0, pl.ds(jnp.clip(t - 4, 0, N_CTX - 1), 1), :])

        @pl.when(t == 9)
        def _(r=r):
            bcast(eot_ref[0, 0:1, :])

        @pl.when(t >= 10)
        def _(r=r, t=t):
            bcast(pad_ref[0, pl.ds(jnp.clip(t - 10, 0, PAD_LEN - 1), 1), :])


def kernel(ctx, emb_sot, emb_cls, emb_eot, emb_pad):
    tposed = pl.pallas_call(
        _body,
        grid=(NSTEPS,),
        in_specs=[
            pl.BlockSpec((1, N_CTX, D), lambda t: (0, 0, 0)),
            pl.BlockSpec((1, 1, D), lambda t: (0, 0, 0)),
            pl.BlockSpec((1, 1, D), lambda t: (0, 0, 0)),
            pl.BlockSpec((1, PAD_LEN, D), lambda t: (0, 0, 0)),
            pl.BlockSpec(memory_space=pltpu.MemorySpace.HBM),
        ],
        out_specs=pl.BlockSpec((SLAB, CLS_NUM, D),
                               lambda s: ((s + 1) % NSTEPS, 0, 0)),
        out_shape=jax.ShapeDtypeStruct((CTX_LEN, CLS_NUM, D), jnp.float32),
        scratch_shapes=[
            pltpu.VMEM((CLS_NUM, N_CLS_TOK, D), jnp.float32),
            pltpu.SemaphoreType.DMA,
        ],
    )(ctx, emb_sot, emb_eot, emb_pad, emb_cls)
    return jnp.transpose(tposed, (1, 0, 2))
